# Initial kernel scaffold; baseline (speedup 1.0000x reference)
import jax
import jax.numpy as jnp
from jax.experimental import pallas as pl


def kernel(x, left_idx, right_idx):
    raise NotImplementedError("write your pallas kernel here")


# speedup vs baseline: 1.0612x; 1.0612x over previous
import functools

import jax
import jax.numpy as jnp
from jax import lax
from jax.experimental import pallas as pl
from jax.experimental.pallas import tpu as pltpu
from jax.experimental.pallas import tpu_sc as plsc

B = 4
IN_SZ = 40962
OUT_SZ = 163842
D = 128
NEW = OUT_SZ - IN_SZ
NC, NS = 2, 16
NW = NC * NS

K = 128
GPW = NEW // NW
GCH = GPW // K
CPW = 1280
CCH = CPW // K
TAIL = IN_SZ - NW * CPW


_mesh = plsc.VectorSubcoreMesh(
    core_axis_name="c", subcore_axis_name="s", num_cores=NC, num_subcores=NS)


_SCRATCH = [
    pltpu.VMEM((K,), jnp.int32),
    pltpu.VMEM((K,), jnp.int32),
    pltpu.VMEM((K,), jnp.int32),
    pltpu.VMEM((K,), jnp.int32),
    pltpu.VMEM((K, D), jnp.float32),
    pltpu.VMEM((K, D), jnp.float32),
    pltpu.SemaphoreType.DMA,
    pltpu.SemaphoreType.DMA,
]


def _upsample_body(x_hbm, li_hbm, ri_hbm, out_hbm,
                   idx_l, idx_r, idx_ls, idx_rs, rows_l, rows_r, sem_l, sem_r):
    wid = lax.axis_index("s") * NC + lax.axis_index("c")

    def copy_chunk(c, carry):
        row = wid * CPW + c * K
        for b in range(B):
            pltpu.sync_copy(x_hbm.at[pl.ds(b * IN_SZ + row, K)], rows_l)
            pltpu.sync_copy(rows_l, out_hbm.at[pl.ds(b * OUT_SZ + row, K)])
        return carry

    lax.fori_loop(0, CCH, copy_chunk, 0)

    @pl.when(wid == 0)
    def _tail():
        base = NW * CPW
        for b in range(B):
            pltpu.sync_copy(x_hbm.at[pl.ds(b * IN_SZ + base, TAIL)],
                            rows_r.at[pl.ds(0, TAIL)])
            pltpu.sync_copy(rows_r.at[pl.ds(0, TAIL)],
                            out_hbm.at[pl.ds(b * OUT_SZ + base, TAIL)])

    def gather_chunk(c, carry):
        base = wid * GPW + c * K
        pltpu.sync_copy(li_hbm.at[pl.ds(base, K)], idx_l)
        pltpu.sync_copy(ri_hbm.at[pl.ds(base, K)], idx_r)
        for b in range(B):
            if b == 0:
                src_l, src_r = idx_l, idx_r
            else:
                off = b * IN_SZ
                for i in range(K // 16):
                    s = pl.ds(i * 16, 16)
                    idx_ls[s] = idx_l[s] + off
                    idx_rs[s] = idx_r[s] + off
                src_l, src_r = idx_ls, idx_rs
            cl = pltpu.async_copy(x_hbm.at[src_l], rows_l, sem_l)
            cr = pltpu.async_copy(x_hbm.at[src_r], rows_r, sem_r)
            cl.wait()
            cr.wait()

            def avg_row(i, carry2):
                for j in range(D // 16):
                    s = pl.ds(j * 16, 16)
                    rows_l[i, s] = (rows_l[i, s] + rows_r[i, s]) * 0.5
                return carry2

            lax.fori_loop(0, K, avg_row, 0)

            pltpu.sync_copy(
                rows_l, out_hbm.at[pl.ds(b * OUT_SZ + IN_SZ + base, K)])
        return carry

    lax.fori_loop(0, GCH, gather_chunk, 0)


_upsample = pl.kernel(
    _upsample_body,
    out_type=jax.ShapeDtypeStruct((B * OUT_SZ, D), jnp.float32),
    mesh=_mesh,
    compiler_params=pltpu.CompilerParams(use_tc_tiling_on_sc=False),
    scratch_types=_SCRATCH,
)


def kernel(x, left_idx, right_idx):
    x_flat = x.reshape(B * IN_SZ, D)
    li = left_idx[IN_SZ:].astype(jnp.int32)
    ri = right_idx[IN_SZ:].astype(jnp.int32)
    out = _upsample(x_flat, li, ri)
    return out.reshape(B, OUT_SZ, D)
